# baseline (speedup 1.0000x reference)
import functools

import jax
import jax.numpy as jnp
from jax import lax
from jax.experimental import pallas as pl
from jax.experimental.pallas import tpu as pltpu
from jax.experimental.pallas import tpu_sc as plsc

N0 = 20000
E = 200000
HEADS = 4
DH = 128
HID = 512
K1 = 10000
K2 = 5000

NP = 20480
E2 = 200704
PADR = NP - N0
RB = 512
NBLK = NP // RB
B1 = 896
B2 = 448
EC = E2 // 2
PT1 = 7
PT2 = 28
ZR = NP // 16
NEG = -1e30


def _mm1_body(x_ref, w_ref, wam_ref, h8_ref, sa_ref):
    xb = x_ref[...]
    h = jnp.dot(xb, w_ref[...], preferred_element_type=jnp.float32)
    for c in range(16):
        h8_ref[c] = h[:, c * 32:(c + 1) * 32]
    sa_ref[...] = jnp.dot(xb, wam_ref[...], preferred_element_type=jnp.float32)


def _mm2_body(h1_ref, s_ref, k_ref, w_ref, wam_ref, h8_ref, sa_ref):
    gate = jnp.tanh(s_ref[...]) * k_ref[...]
    xb = jnp.where(k_ref[...] > 0, h1_ref[...] * gate, 0.0)
    h = jnp.dot(xb, w_ref[...], preferred_element_type=jnp.float32)
    for c in range(16):
        h8_ref[c] = h[:, c * 32:(c + 1) * 32]
    sa = jnp.dot(xb, wam_ref[...], preferred_element_type=jnp.float32)
    sa_ref[...] = jnp.where(k_ref[...] > 0, sa, NEG)


def _combine_body(acc8_ref, h8_ref, sa_ref, den_ref, b_ref, wpn_ref, keep_ref,
                  h_ref, score_ref):
    i = pl.program_id(0)
    sa = sa_ref[...]
    pre = sa[:, 0:4] + sa[:, 4:8]
    eself = jnp.exp(jnp.where(pre >= 0, pre, 0.2 * pre))
    dent = jnp.maximum(den_ref[...] + eself, 1e-20)
    cols = []
    for c in range(16):
        hd = c // 4
        es = eself[:, hd:hd + 1]
        dn = dent[:, hd:hd + 1]
        v = (acc8_ref[c] + h8_ref[c] * es) / dn + b_ref[0, c * 32:(c + 1) * 32]
        hv = jnp.where(v > 0, v, (jnp.exp(v) - 1.0))
        h_ref[:, c * 32:(c + 1) * 32] = hv
        cols.append(hv)
    hb = jnp.concatenate(cols, axis=1)
    sc = jnp.dot(hb, wpn_ref[...], preferred_element_type=jnp.float32)
    rid = i * RB + lax.broadcasted_iota(jnp.int32, (RB, 1), 0)
    valid = (rid < N0) & (keep_ref[...] > 0)
    score_ref[...] = jnp.where(valid, sc, NEG)


def _select_body(s_ref, keep_ref, *, k):
    s = s_ref[...]
    bits = lax.bitcast_convert_type(s, jnp.int32)
    key = jnp.where(bits >= 0, bits, bits ^ jnp.int32(0x7FFFFFFF))
    c_nonneg = jnp.sum((key >= 0).astype(jnp.int32))
    t0 = jnp.where(c_nonneg >= k, jnp.int32(0), jnp.int32(-2147483648))

    def body(j, t):
        tb = t | lax.shift_left(jnp.int32(1), 30 - j)
        cnt = jnp.sum((key >= tb).astype(jnp.int32))
        return jnp.where(cnt >= k, tb, t)

    t = lax.fori_loop(0, 31, body, t0)
    keep_ref[...] = (key >= t).astype(jnp.float32)


def _final_body(h_ref, s_ref, k_ref, wl_ref, bl_ref, o_ref, acc_ref):
    i = pl.program_id(0)

    @pl.when(i == 0)
    def _():
        acc_ref[...] = jnp.zeros_like(acc_ref)

    gate = jnp.tanh(s_ref[...]) * k_ref[...]
    masked = jnp.where(k_ref[...] > 0, h_ref[...] * gate, 0.0)
    acc_ref[...] += jnp.sum(masked, axis=0, keepdims=True)

    @pl.when(i == pl.num_programs(0) - 1)
    def _():
        g = acc_ref[...] / jnp.float32(K2)
        o_ref[...] = jnp.dot(g, wl_ref[...],
                             preferred_element_type=jnp.float32) + bl_ref[...]


def _mm1(xpad, W, Wam):
    return pl.pallas_call(
        _mm1_body,
        grid=(NBLK,),
        in_specs=[
            pl.BlockSpec((RB, 64), lambda i: (i, 0)),
            pl.BlockSpec((64, HID), lambda i: (0, 0)),
            pl.BlockSpec((64, 8), lambda i: (0, 0)),
        ],
        out_specs=[
            pl.BlockSpec((16, RB, 32), lambda i: (0, i, 0)),
            pl.BlockSpec((RB, 8), lambda i: (i, 0)),
        ],
        out_shape=[
            jax.ShapeDtypeStruct((16, NP, 32), jnp.float32),
            jax.ShapeDtypeStruct((NP, 8), jnp.float32),
        ],
    )(xpad, W, Wam)


def _mm2(h1, score1, keep1, W, Wam):
    return pl.pallas_call(
        _mm2_body,
        grid=(NBLK,),
        in_specs=[
            pl.BlockSpec((RB, HID), lambda i: (i, 0)),
            pl.BlockSpec((RB, 1), lambda i: (i, 0)),
            pl.BlockSpec((RB, 1), lambda i: (i, 0)),
            pl.BlockSpec((HID, HID), lambda i: (0, 0)),
            pl.BlockSpec((HID, 8), lambda i: (0, 0)),
        ],
        out_specs=[
            pl.BlockSpec((16, RB, 32), lambda i: (0, i, 0)),
            pl.BlockSpec((RB, 8), lambda i: (i, 0)),
        ],
        out_shape=[
            jax.ShapeDtypeStruct((16, NP, 32), jnp.float32),
            jax.ShapeDtypeStruct((NP, 8), jnp.float32),
        ],
    )(h1, score1, keep1, W, Wam)


def _combine(acc8, h8, sa, den_n, brow, wpn, keep):
    return pl.pallas_call(
        _combine_body,
        grid=(NBLK,),
        in_specs=[
            pl.BlockSpec((16, RB, 32), lambda i: (0, i, 0)),
            pl.BlockSpec((16, RB, 32), lambda i: (0, i, 0)),
            pl.BlockSpec((RB, 8), lambda i: (i, 0)),
            pl.BlockSpec((RB, 4), lambda i: (i, 0)),
            pl.BlockSpec((1, HID), lambda i: (0, 0)),
            pl.BlockSpec((HID, 1), lambda i: (0, 0)),
            pl.BlockSpec((RB, 1), lambda i: (i, 0)),
        ],
        out_specs=[
            pl.BlockSpec((RB, HID), lambda i: (i, 0)),
            pl.BlockSpec((RB, 1), lambda i: (i, 0)),
        ],
        out_shape=[
            jax.ShapeDtypeStruct((NP, HID), jnp.float32),
            jax.ShapeDtypeStruct((NP, 1), jnp.float32),
        ],
    )(acc8, h8, sa, den_n, brow, wpn, keep)


def _select(score, k):
    s2 = score.reshape(NP // 128, 128)
    keep = pl.pallas_call(
        functools.partial(_select_body, k=k),
        out_shape=jax.ShapeDtypeStruct((NP // 128, 128), jnp.float32),
    )(s2)
    return keep.reshape(NP, 1)


def _final(h2o, score2, keep2, Wl, blrow):
    return pl.pallas_call(
        _final_body,
        grid=(NBLK,),
        in_specs=[
            pl.BlockSpec((RB, HID), lambda i: (i, 0)),
            pl.BlockSpec((RB, 1), lambda i: (i, 0)),
            pl.BlockSpec((RB, 1), lambda i: (i, 0)),
            pl.BlockSpec((HID, 10), lambda i: (0, 0)),
            pl.BlockSpec((1, 10), lambda i: (0, 0)),
        ],
        out_specs=pl.BlockSpec((1, 10), lambda i: (0, 0)),
        out_shape=jax.ShapeDtypeStruct((1, 10), jnp.float32),
        scratch_shapes=[pltpu.VMEM((1, HID), jnp.float32)],
    )(h2o, score2, keep2, Wl, blrow)


def _mesh():
    return plsc.VectorSubcoreMesh(core_axis_name="c", subcore_axis_name="s")


def _sc_pass1_body(src_ref, dst_ref, sas_ref, sad_ref, ee_ref, den_ref,
                   sidx, didx, sval, dval, eeb, zbuf, den_sh):
    cid = lax.axis_index("c")
    sid = lax.axis_index("s")

    def zb(i, _):
        zbuf[i, :] = jnp.zeros((16,), jnp.float32)
        return 0

    lax.fori_loop(0, ZR, zb, 0)
    pltpu.sync_copy(zbuf, den_sh.at[pl.ds(sid * ZR, ZR)])
    plsc.subcore_barrier()

    for t in range(PT1):
        base = cid * EC + sid * (EC // 16) + t * B1
        pltpu.sync_copy(src_ref.at[pl.ds(base, B1)], sidx)
        pltpu.sync_copy(dst_ref.at[pl.ds(base, B1)], didx)
        pltpu.sync_copy(sas_ref.at[sidx], sval)
        pltpu.sync_copy(sad_ref.at[didx], dval)

        def cb(i, _):
            a = sval[i, :] + dval[i, :]
            l = jnp.where(a >= 0, a, 0.2 * a)
            eeb[i, :] = jnp.exp(l)
            return 0

        lax.fori_loop(0, B1, cb, 0)

        pltpu.sync_copy(eeb, ee_ref.at[pl.ds(base, B1)])
        pltpu.sync_copy(eeb, den_sh.at[didx], add=True)

    plsc.subcore_barrier()
    pltpu.sync_copy(den_sh.at[pl.ds(sid * ZR, ZR)], zbuf)
    pltpu.sync_copy(zbuf, den_ref.at[cid, pl.ds(sid * ZR, ZR)])


def _sc_pass1(srcp, dstp, sas16, sad16):
    f = pl.kernel(
        _sc_pass1_body,
        out_type=[
            jax.ShapeDtypeStruct((E2, 16), jnp.float32),
            jax.ShapeDtypeStruct((2, NP, 16), jnp.float32),
        ],
        mesh=_mesh(),
        compiler_params=pltpu.CompilerParams(use_tc_tiling_on_sc=False),
        scratch_types=[
            pltpu.VMEM((B1,), jnp.int32),
            pltpu.VMEM((B1,), jnp.int32),
            pltpu.VMEM((B1, 16), jnp.float32),
            pltpu.VMEM((B1, 16), jnp.float32),
            pltpu.VMEM((B1, 16), jnp.float32),
            pltpu.VMEM((ZR, 16), jnp.float32),
            pltpu.VMEM_SHARED((NP, 16), jnp.float32),
        ],
    )
    return f(srcp, dstp, sas16, sad16)


def _sc_pass2_body(src_ref, dst_ref, ee_ref, h8_ref, acc8_ref,
                   sidx0, sidx1, didx0, didx1, eec0, eec1, rows0, rows1,
                   wb, acc_sh, gsem):
    sidx = (sidx0, sidx1)
    didx = (didx0, didx1)
    eec = (eec0, eec1)
    rows = (rows0, rows1)
    cid = lax.axis_index("c")
    sid = lax.axis_index("s")

    if True:
        for cc in range(8):
            c = cc * 2 + cid

            def zb(i, _):
                for j in range(2):
                    wb[i, pl.ds(j * 16, 16)] = jnp.zeros((16,), jnp.float32)
                return 0

            lax.fori_loop(0, ZR // 2, zb, 0)
            for half in range(2):
                pltpu.sync_copy(
                    wb, acc_sh.at[pl.ds(sid * ZR + half * (ZR // 2), ZR // 2)])
            plsc.subcore_barrier()

            def load_idx(base, s):
                pltpu.sync_copy(src_ref.at[pl.ds(base, B2)], sidx[s])
                pltpu.sync_copy(dst_ref.at[pl.ds(base, B2)], didx[s])
                pltpu.sync_copy(ee_ref.at[pl.ds(base, B2)], eec[s])

            def scale(s, cc):
                def sc(i, _):
                    ev = eec[s][i, :][cc // 2]
                    for j in range(2):
                        rows[s][i, pl.ds(j * 16, 16)] = (
                            rows[s][i, pl.ds(j * 16, 16)] * ev)
                    return 0

                lax.fori_loop(0, B2, sc, 0, unroll=8)

            def tt_body(tt, _, cc=cc, c=c):
                base = sid * (E2 // 16) + (2 * tt) * B2
                load_idx(base, 0)
                pltpu.sync_copy(h8_ref.at[c].at[sidx[0]], rows[0])
                load_idx(base + B2, 1)
                gb = pltpu.async_copy(h8_ref.at[c].at[sidx[1]],
                                      rows[1], gsem.at[1])
                scale(0, cc)
                pltpu.sync_copy(rows[0], acc_sh.at[didx[0]], add=True)
                gb.wait()
                scale(1, cc)
                pltpu.sync_copy(rows[1], acc_sh.at[didx[1]], add=True)
                return 0

            lax.fori_loop(0, PT2 // 2, tt_body, 0)

            plsc.subcore_barrier()
            for half in range(2):
                r0 = sid * ZR + half * (ZR // 2)
                pltpu.sync_copy(acc_sh.at[pl.ds(r0, ZR // 2)], wb)
                pltpu.sync_copy(wb, acc8_ref.at[c, pl.ds(r0, ZR // 2)])
            plsc.subcore_barrier()


def _sc_pass2(srcp, dstp, ee, h8):
    f = pl.kernel(
        _sc_pass2_body,
        out_type=jax.ShapeDtypeStruct((16, NP, 32), jnp.float32),
        mesh=_mesh(),
        compiler_params=pltpu.CompilerParams(use_tc_tiling_on_sc=False),
        scratch_types=[
            pltpu.VMEM((B2,), jnp.int32),
            pltpu.VMEM((B2,), jnp.int32),
            pltpu.VMEM((B2,), jnp.int32),
            pltpu.VMEM((B2,), jnp.int32),
            pltpu.VMEM((B2, 16), jnp.float32),
            pltpu.VMEM((B2, 16), jnp.float32),
            pltpu.VMEM((B2, 32), jnp.float32),
            pltpu.VMEM((B2, 32), jnp.float32),
            pltpu.VMEM((ZR // 2, 32), jnp.float32),
            pltpu.VMEM_SHARED((NP, 32), jnp.float32),
            pltpu.SemaphoreType.DMA((2,)),
        ],
    )
    return f(srcp, dstp, ee, h8)


def kernel(x, edge_index, batch, W1, a_src1, a_dst1, b1, wp1, W2, a_src2,
           a_dst2, b2, wp2, Wl, bl):
    del batch
    f32 = jnp.float32

    xpad = jnp.pad(x, ((0, NP - N0), (0, 0)))
    dummy = (N0 + (jnp.arange(E2 - E, dtype=jnp.int32) % PADR))
    srcp = jnp.concatenate([edge_index[0], dummy])
    dstp = jnp.concatenate([edge_index[1], dummy])
    blkdiag = jnp.kron(jnp.eye(HEADS, dtype=f32), jnp.ones((DH, 1), f32))

    def fold(W, a_s, a_d):
        am = jnp.concatenate([blkdiag * a_s.reshape(HID)[:, None],
                              blkdiag * a_d.reshape(HID)[:, None]], axis=1)
        return W @ am

    Wam1 = fold(W1, a_src1, a_dst1)
    Wam2 = fold(W2, a_src2, a_dst2)
    wpn1 = (wp1 / jnp.linalg.norm(wp1)).reshape(HID, 1)
    wpn2 = (wp2 / jnp.linalg.norm(wp2)).reshape(HID, 1)
    b1r = b1.reshape(1, HID)
    b2r = b2.reshape(1, HID)
    blr = bl.reshape(1, 10)
    ones = jnp.ones((NP, 1), f32)

    def widen(sa):
        return (jnp.pad(sa[:, 0:4], ((0, 0), (0, 12))),
                jnp.pad(sa[:, 4:8], ((0, 0), (0, 12))))

    h8_1, sa1 = _mm1(xpad, W1, Wam1)
    sas1, sad1 = widen(sa1)
    ee1, den1 = _sc_pass1(srcp, dstp, sas1, sad1)
    acc8_1 = _sc_pass2(srcp, dstp, ee1, h8_1)
    h1, score1 = _combine(acc8_1, h8_1, sa1, den1.sum(0)[:, 0:4], b1r, wpn1,
                          ones)
    keep1 = _select(score1, K1)

    h8_2, sa2 = _mm2(h1, score1, keep1, W2, Wam2)
    sas2, sad2 = widen(sa2)
    ee2, den2 = _sc_pass1(srcp, dstp, sas2, sad2)
    acc8_2 = _sc_pass2(srcp, dstp, ee2, h8_2)
    h2o, score2 = _combine(acc8_2, h8_2, sa2, den2.sum(0)[:, 0:4], b2r, wpn2,
                           keep1)
    keep2 = _select(score2, K2)

    return _final(h2o, score2, keep2, Wl, blr)


# speedup vs baseline: 1.1409x; 1.1409x over previous
import functools

import jax
import jax.numpy as jnp
from jax import lax
from jax.experimental import pallas as pl
from jax.experimental.pallas import tpu as pltpu
from jax.experimental.pallas import tpu_sc as plsc

N0 = 20000
E = 200000
HEADS = 4
DH = 128
HID = 512
K1 = 10000
K2 = 5000

NP = 20480
E2 = 200704
PADR = NP - N0
RB = 512
NBLK = NP // RB
B1 = 896
B2 = 896
EC = E2 // 2
PT1 = 7
PT2 = 14
ZR = NP // 16
NEG = -1e30


def _mm1_body(x_ref, w_ref, wam_ref, h8_ref, sa_ref):
    xb = x_ref[...]
    h = jnp.dot(xb, w_ref[...], preferred_element_type=jnp.float32)
    for c in range(16):
        h8_ref[c] = h[:, c * 32:(c + 1) * 32]
    sa_ref[...] = jnp.dot(xb, wam_ref[...], preferred_element_type=jnp.float32)


def _mm2_body(h1_ref, s_ref, k_ref, w_ref, wam_ref, h8_ref, sa_ref):
    gate = jnp.tanh(s_ref[...]) * k_ref[...]
    xb = jnp.where(k_ref[...] > 0, h1_ref[...] * gate, 0.0)
    h = jnp.dot(xb, w_ref[...], preferred_element_type=jnp.float32)
    for c in range(16):
        h8_ref[c] = h[:, c * 32:(c + 1) * 32]
    sa = jnp.dot(xb, wam_ref[...], preferred_element_type=jnp.float32)
    sa_ref[...] = jnp.where(k_ref[...] > 0, sa, NEG)


def _combine_body(acc8_ref, h8_ref, sa_ref, den_ref, b_ref, wpn_ref, keep_ref,
                  h_ref, score_ref):
    i = pl.program_id(0)
    sa = sa_ref[...]
    pre = sa[:, 0:4] + sa[:, 4:8]
    eself = jnp.exp(jnp.where(pre >= 0, pre, 0.2 * pre))
    dent = jnp.maximum(den_ref[...] + eself, 1e-20)
    cols = []
    for c in range(16):
        hd = c // 4
        es = eself[:, hd:hd + 1]
        dn = dent[:, hd:hd + 1]
        v = (acc8_ref[c] + h8_ref[c] * es) / dn + b_ref[0, c * 32:(c + 1) * 32]
        hv = jnp.where(v > 0, v, (jnp.exp(v) - 1.0))
        h_ref[:, c * 32:(c + 1) * 32] = hv
        cols.append(hv)
    hb = jnp.concatenate(cols, axis=1)
    sc = jnp.dot(hb, wpn_ref[...], preferred_element_type=jnp.float32)
    rid = i * RB + lax.broadcasted_iota(jnp.int32, (RB, 1), 0)
    valid = (rid < N0) & (keep_ref[...] > 0)
    score_ref[...] = jnp.where(valid, sc, NEG)


def _select_body(s_ref, keep_ref, *, k):
    s = s_ref[...]
    bits = lax.bitcast_convert_type(s, jnp.int32)
    key = jnp.where(bits >= 0, bits, bits ^ jnp.int32(0x7FFFFFFF))
    c_nonneg = jnp.sum((key >= 0).astype(jnp.int32))
    t0 = jnp.where(c_nonneg >= k, jnp.int32(0), jnp.int32(-2147483648))

    def body(j, t):
        tb = t | lax.shift_left(jnp.int32(1), 30 - j)
        cnt = jnp.sum((key >= tb).astype(jnp.int32))
        return jnp.where(cnt >= k, tb, t)

    t = lax.fori_loop(0, 31, body, t0)
    keep_ref[...] = (key >= t).astype(jnp.float32)


def _final_body(h_ref, s_ref, k_ref, wl_ref, bl_ref, o_ref, acc_ref):
    i = pl.program_id(0)

    @pl.when(i == 0)
    def _():
        acc_ref[...] = jnp.zeros_like(acc_ref)

    gate = jnp.tanh(s_ref[...]) * k_ref[...]
    masked = jnp.where(k_ref[...] > 0, h_ref[...] * gate, 0.0)
    acc_ref[...] += jnp.sum(masked, axis=0, keepdims=True)

    @pl.when(i == pl.num_programs(0) - 1)
    def _():
        g = acc_ref[...] / jnp.float32(K2)
        o_ref[...] = jnp.dot(g, wl_ref[...],
                             preferred_element_type=jnp.float32) + bl_ref[...]


def _mm1(xpad, W, Wam):
    return pl.pallas_call(
        _mm1_body,
        grid=(NBLK,),
        in_specs=[
            pl.BlockSpec((RB, 64), lambda i: (i, 0)),
            pl.BlockSpec((64, HID), lambda i: (0, 0)),
            pl.BlockSpec((64, 8), lambda i: (0, 0)),
        ],
        out_specs=[
            pl.BlockSpec((16, RB, 32), lambda i: (0, i, 0)),
            pl.BlockSpec((RB, 8), lambda i: (i, 0)),
        ],
        out_shape=[
            jax.ShapeDtypeStruct((16, NP, 32), jnp.float32),
            jax.ShapeDtypeStruct((NP, 8), jnp.float32),
        ],
    )(xpad, W, Wam)


def _mm2(h1, score1, keep1, W, Wam):
    return pl.pallas_call(
        _mm2_body,
        grid=(NBLK,),
        in_specs=[
            pl.BlockSpec((RB, HID), lambda i: (i, 0)),
            pl.BlockSpec((RB, 1), lambda i: (i, 0)),
            pl.BlockSpec((RB, 1), lambda i: (i, 0)),
            pl.BlockSpec((HID, HID), lambda i: (0, 0)),
            pl.BlockSpec((HID, 8), lambda i: (0, 0)),
        ],
        out_specs=[
            pl.BlockSpec((16, RB, 32), lambda i: (0, i, 0)),
            pl.BlockSpec((RB, 8), lambda i: (i, 0)),
        ],
        out_shape=[
            jax.ShapeDtypeStruct((16, NP, 32), jnp.float32),
            jax.ShapeDtypeStruct((NP, 8), jnp.float32),
        ],
    )(h1, score1, keep1, W, Wam)


def _combine(acc8, h8, sa, den_n, brow, wpn, keep):
    return pl.pallas_call(
        _combine_body,
        grid=(NBLK,),
        in_specs=[
            pl.BlockSpec((16, RB, 32), lambda i: (0, i, 0)),
            pl.BlockSpec((16, RB, 32), lambda i: (0, i, 0)),
            pl.BlockSpec((RB, 8), lambda i: (i, 0)),
            pl.BlockSpec((RB, 4), lambda i: (i, 0)),
            pl.BlockSpec((1, HID), lambda i: (0, 0)),
            pl.BlockSpec((HID, 1), lambda i: (0, 0)),
            pl.BlockSpec((RB, 1), lambda i: (i, 0)),
        ],
        out_specs=[
            pl.BlockSpec((RB, HID), lambda i: (i, 0)),
            pl.BlockSpec((RB, 1), lambda i: (i, 0)),
        ],
        out_shape=[
            jax.ShapeDtypeStruct((NP, HID), jnp.float32),
            jax.ShapeDtypeStruct((NP, 1), jnp.float32),
        ],
    )(acc8, h8, sa, den_n, brow, wpn, keep)


def _select(score, k):
    s2 = score.reshape(NP // 128, 128)
    keep = pl.pallas_call(
        functools.partial(_select_body, k=k),
        out_shape=jax.ShapeDtypeStruct((NP // 128, 128), jnp.float32),
    )(s2)
    return keep.reshape(NP, 1)


def _final(h2o, score2, keep2, Wl, blrow):
    return pl.pallas_call(
        _final_body,
        grid=(NBLK,),
        in_specs=[
            pl.BlockSpec((RB, HID), lambda i: (i, 0)),
            pl.BlockSpec((RB, 1), lambda i: (i, 0)),
            pl.BlockSpec((RB, 1), lambda i: (i, 0)),
            pl.BlockSpec((HID, 10), lambda i: (0, 0)),
            pl.BlockSpec((1, 10), lambda i: (0, 0)),
        ],
        out_specs=pl.BlockSpec((1, 10), lambda i: (0, 0)),
        out_shape=jax.ShapeDtypeStruct((1, 10), jnp.float32),
        scratch_shapes=[pltpu.VMEM((1, HID), jnp.float32)],
    )(h2o, score2, keep2, Wl, blrow)


def _mesh():
    return plsc.VectorSubcoreMesh(core_axis_name="c", subcore_axis_name="s")


def _sc_pass1_body(src_ref, dst_ref, sas_ref, sad_ref, ee_ref, den_ref,
                   sidx, didx, sval, dval, eeb, zbuf, den_sh):
    cid = lax.axis_index("c")
    sid = lax.axis_index("s")

    def zb(i, _):
        zbuf[i, :] = jnp.zeros((16,), jnp.float32)
        return 0

    lax.fori_loop(0, ZR, zb, 0)
    pltpu.sync_copy(zbuf, den_sh.at[pl.ds(sid * ZR, ZR)])
    plsc.subcore_barrier()

    for t in range(PT1):
        base = cid * EC + sid * (EC // 16) + t * B1
        pltpu.sync_copy(src_ref.at[pl.ds(base, B1)], sidx)
        pltpu.sync_copy(dst_ref.at[pl.ds(base, B1)], didx)
        pltpu.sync_copy(sas_ref.at[sidx], sval)
        pltpu.sync_copy(sad_ref.at[didx], dval)

        def cb(i, _):
            a = sval[i, :] + dval[i, :]
            l = jnp.where(a >= 0, a, 0.2 * a)
            eeb[i, :] = jnp.exp(l)
            return 0

        lax.fori_loop(0, B1, cb, 0)

        pltpu.sync_copy(eeb, ee_ref.at[pl.ds(base, B1)])
        pltpu.sync_copy(eeb, den_sh.at[didx], add=True)

    plsc.subcore_barrier()
    pltpu.sync_copy(den_sh.at[pl.ds(sid * ZR, ZR)], zbuf)
    pltpu.sync_copy(zbuf, den_ref.at[cid, pl.ds(sid * ZR, ZR)])


def _sc_pass1(srcp, dstp, sas16, sad16):
    f = pl.kernel(
        _sc_pass1_body,
        out_type=[
            jax.ShapeDtypeStruct((E2, 16), jnp.float32),
            jax.ShapeDtypeStruct((2, NP, 16), jnp.float32),
        ],
        mesh=_mesh(),
        compiler_params=pltpu.CompilerParams(use_tc_tiling_on_sc=False),
        scratch_types=[
            pltpu.VMEM((B1,), jnp.int32),
            pltpu.VMEM((B1,), jnp.int32),
            pltpu.VMEM((B1, 16), jnp.float32),
            pltpu.VMEM((B1, 16), jnp.float32),
            pltpu.VMEM((B1, 16), jnp.float32),
            pltpu.VMEM((ZR, 16), jnp.float32),
            pltpu.VMEM_SHARED((NP, 16), jnp.float32),
        ],
    )
    return f(srcp, dstp, sas16, sad16)


def _sc_pass2_body(src_ref, dst_ref, ee_ref, h8_ref, acc8_ref,
                   sidx0, sidx1, didx0, didx1, eec0, rows0, rows1,
                   acc_sh, gsem):
    sidx = (sidx0, sidx1)
    didx = (didx0, didx1)
    rows = (rows0, rows1)
    cid = lax.axis_index("c")
    sid = lax.axis_index("s")

    if True:
        for cc in range(8):
            c = cc * 2 + cid

            def zb(i, _):
                for j in range(2):
                    rows0[i, pl.ds(j * 16, 16)] = jnp.zeros((16,), jnp.float32)
                return 0

            lax.fori_loop(0, ZR // 2, zb, 0)
            for half in range(2):
                pltpu.sync_copy(
                    rows0.at[pl.ds(0, ZR // 2)],
                    acc_sh.at[pl.ds(sid * ZR + half * (ZR // 2), ZR // 2)])
            plsc.subcore_barrier()

            def load_idx(base, s):
                pltpu.sync_copy(src_ref.at[pl.ds(base, B2)], sidx[s])
                pltpu.sync_copy(dst_ref.at[pl.ds(base, B2)], didx[s])

            def scale(s, cc, base):
                pltpu.sync_copy(ee_ref.at[pl.ds(base, B2)], eec0)

                def sc(i, _):
                    ev = eec0[i, :][cc // 2]
                    for j in range(2):
                        rows[s][i, pl.ds(j * 16, 16)] = (
                            rows[s][i, pl.ds(j * 16, 16)] * ev)
                    return 0

                lax.fori_loop(0, B2, sc, 0, unroll=8)

            def tt_body(tt, _, cc=cc, c=c):
                base = sid * (E2 // 16) + (2 * tt) * B2
                load_idx(base, 0)
                ga = pltpu.async_copy(h8_ref.at[c].at[sidx[0]],
                                      rows[0], gsem.at[0])
                load_idx(base + B2, 1)
                gb = pltpu.async_copy(h8_ref.at[c].at[sidx[1]],
                                      rows[1], gsem.at[1])
                ga.wait()
                scale(0, cc, base)
                pltpu.sync_copy(rows[0], acc_sh.at[didx[0]], add=True)
                gb.wait()
                scale(1, cc, base + B2)
                pltpu.sync_copy(rows[1], acc_sh.at[didx[1]], add=True)
                return 0

            lax.fori_loop(0, PT2 // 2, tt_body, 0)

            plsc.subcore_barrier()
            for half in range(2):
                r0 = sid * ZR + half * (ZR // 2)
                pltpu.sync_copy(acc_sh.at[pl.ds(r0, ZR // 2)],
                                rows0.at[pl.ds(0, ZR // 2)])
                pltpu.sync_copy(rows0.at[pl.ds(0, ZR // 2)],
                                acc8_ref.at[c, pl.ds(r0, ZR // 2)])
            plsc.subcore_barrier()


def _sc_pass2(srcp, dstp, ee, h8):
    f = pl.kernel(
        _sc_pass2_body,
        out_type=jax.ShapeDtypeStruct((16, NP, 32), jnp.float32),
        mesh=_mesh(),
        compiler_params=pltpu.CompilerParams(use_tc_tiling_on_sc=False),
        scratch_types=[
            pltpu.VMEM((B2,), jnp.int32),
            pltpu.VMEM((B2,), jnp.int32),
            pltpu.VMEM((B2,), jnp.int32),
            pltpu.VMEM((B2,), jnp.int32),
            pltpu.VMEM((B2, 16), jnp.float32),
            pltpu.VMEM((B2, 32), jnp.float32),
            pltpu.VMEM((B2, 32), jnp.float32),
            pltpu.VMEM_SHARED((NP, 32), jnp.float32),
            pltpu.SemaphoreType.DMA((2,)),
        ],
    )
    return f(srcp, dstp, ee, h8)


def kernel(x, edge_index, batch, W1, a_src1, a_dst1, b1, wp1, W2, a_src2,
           a_dst2, b2, wp2, Wl, bl):
    del batch
    f32 = jnp.float32

    xpad = jnp.pad(x, ((0, NP - N0), (0, 0)))
    dummy = (N0 + (jnp.arange(E2 - E, dtype=jnp.int32) % PADR))
    srcp = jnp.concatenate([edge_index[0], dummy])
    dstp = jnp.concatenate([edge_index[1], dummy])
    blkdiag = jnp.kron(jnp.eye(HEADS, dtype=f32), jnp.ones((DH, 1), f32))

    def fold(W, a_s, a_d):
        am = jnp.concatenate([blkdiag * a_s.reshape(HID)[:, None],
                              blkdiag * a_d.reshape(HID)[:, None]], axis=1)
        return W @ am

    Wam1 = fold(W1, a_src1, a_dst1)
    Wam2 = fold(W2, a_src2, a_dst2)
    wpn1 = (wp1 / jnp.linalg.norm(wp1)).reshape(HID, 1)
    wpn2 = (wp2 / jnp.linalg.norm(wp2)).reshape(HID, 1)
    b1r = b1.reshape(1, HID)
    b2r = b2.reshape(1, HID)
    blr = bl.reshape(1, 10)
    ones = jnp.ones((NP, 1), f32)

    def widen(sa):
        return (jnp.pad(sa[:, 0:4], ((0, 0), (0, 12))),
                jnp.pad(sa[:, 4:8], ((0, 0), (0, 12))))

    h8_1, sa1 = _mm1(xpad, W1, Wam1)
    sas1, sad1 = widen(sa1)
    ee1, den1 = _sc_pass1(srcp, dstp, sas1, sad1)
    acc8_1 = _sc_pass2(srcp, dstp, ee1, h8_1)
    h1, score1 = _combine(acc8_1, h8_1, sa1, den1.sum(0)[:, 0:4], b1r, wpn1,
                          ones)
    keep1 = _select(score1, K1)

    h8_2, sa2 = _mm2(h1, score1, keep1, W2, Wam2)
    sas2, sad2 = widen(sa2)
    ee2, den2 = _sc_pass1(srcp, dstp, sas2, sad2)
    acc8_2 = _sc_pass2(srcp, dstp, ee2, h8_2)
    h2o, score2 = _combine(acc8_2, h8_2, sa2, den2.sum(0)[:, 0:4], b2r, wpn2,
                           keep1)
    keep2 = _select(score2, K2)

    return _final(h2o, score2, keep2, Wl, blr)
